# Initial kernel scaffold; baseline (speedup 1.0000x reference)
import jax
import jax.numpy as jnp
from jax.experimental import pallas as pl


def kernel(points, features, W):
    raise NotImplementedError("write your pallas kernel here")


# speedup vs baseline: 6.6223x; 6.6223x over previous
import functools

import jax
import jax.numpy as jnp
from jax import lax
from jax.experimental import pallas as pl
from jax.experimental.pallas import tpu as pltpu
from jax.experimental.pallas import tpu_sc as plsc

_R2 = 0.2 * 0.2
_NS = 32
_N = 4096
_NP = 1024
_B = 2
_CIN = 64
_COUT = 64
_DP = 128
_NC, _NSUB = 2, 16
_NW = _NC * _NSUB
_CPW = (_B * _NP) // _NW


def _fps_body(x0_ref, y0_ref, z0_ref, x1_ref, y1_ref, z1_ref,
              nx_ref, ny_ref, nz_ref):
    row = lax.broadcasted_iota(jnp.int32, (32, 128), 0)
    col = lax.broadcasted_iota(jnp.int32, (32, 128), 1)
    gid = row * 128 + col
    planes = ((x0_ref[...], y0_ref[...], z0_ref[...]),
              (x1_ref[...], y1_ref[...], z1_ref[...]))
    outs = (nx_ref, ny_ref, nz_ref)

    def pick(arr, sel):
        return jnp.sum(jnp.where(sel, arr, 0.0))

    def body(i, carry):
        d0, l0, d1, l1 = carry
        new = []
        for b, (d, l) in enumerate(((d0, l0), (d1, l1))):
            x, y, z = planes[b]
            sel = gid == l
            cx, cy, cz = pick(x, sel), pick(y, sel), pick(z, sel)
            outs[0][b, i - 1] = cx
            outs[1][b, i - 1] = cy
            outs[2][b, i - 1] = cz
            dx, dy, dz = x - cx, y - cy, z - cz
            dd = dx * dx + dy * dy + dz * dz
            d = jnp.minimum(d, dd)
            m = jnp.max(d)
            l = jnp.min(jnp.where(d == m, gid, _N))
            new.append((d, l))
        return (new[0][0], new[0][1], new[1][0], new[1][1])

    init = (jnp.full((32, 128), 1e10, jnp.float32), jnp.int32(0),
            jnp.full((32, 128), 1e10, jnp.float32), jnp.int32(0))
    d0, l0, d1, l1 = lax.fori_loop(1, _NP, body, init)
    for b, l in enumerate((l0, l1)):
        x, y, z = planes[b]
        sel = gid == l
        outs[0][b, _NP - 1] = pick(x, sel)
        outs[1][b, _NP - 1] = pick(y, sel)
        outs[2][b, _NP - 1] = pick(z, sel)


def _fps(points):
    pts = jnp.transpose(points, (0, 2, 1))
    blocks = [pts[b, d].reshape(32, 128) for b in range(_B) for d in range(3)]
    args = [blocks[0], blocks[1], blocks[2], blocks[3], blocks[4], blocks[5]]
    out = pl.pallas_call(
        _fps_body,
        out_shape=[jax.ShapeDtypeStruct((_B, _NP), jnp.float32)] * 3,
        in_specs=[pl.BlockSpec(memory_space=pltpu.VMEM)] * 6,
        out_specs=[pl.BlockSpec(memory_space=pltpu.SMEM)] * 3,
    )(*args)
    return out


@functools.cache
def _sc_mesh():
    return plsc.VectorSubcoreMesh(core_axis_name="c", subcore_axis_name="s",
                                  num_cores=_NC, num_subcores=_NSUB)


@functools.cache
def _build_ballq():
  return pl.kernel(
    _ballq_body,
    out_type=jax.ShapeDtypeStruct((_B * _NP, _NS), jnp.int32),
    mesh=_sc_mesh(),
    compiler_params=pltpu.CompilerParams(needs_layout_passes=False),
    scratch_types=[
        pltpu.VMEM((_N,), jnp.float32),
        pltpu.VMEM((_N,), jnp.float32),
        pltpu.VMEM((_N,), jnp.float32),
        pltpu.VMEM((_CPW,), jnp.float32),
        pltpu.VMEM((_CPW,), jnp.float32),
        pltpu.VMEM((_CPW,), jnp.float32),
        pltpu.VMEM((48,), jnp.int32),
        pltpu.VMEM((_NS,), jnp.int32),
    ],
  )


def _ballq_body(xf, yf, zf, nxf, nyf, nzf, idx_hbm,
                xw, yw, zw, cxw, cyw, czw, buf, rowbuf):
    wid = lax.axis_index("s") * _NC + lax.axis_index("c")
    b = wid // _NSUB
    cbase = wid * _CPW
    pltpu.sync_copy(xf.at[pl.ds(b * _N, _N)], xw)
    pltpu.sync_copy(yf.at[pl.ds(b * _N, _N)], yw)
    pltpu.sync_copy(zf.at[pl.ds(b * _N, _N)], zw)
    pltpu.sync_copy(nxf.at[pl.ds(cbase, _CPW)], cxw)
    pltpu.sync_copy(nyf.at[pl.ds(cbase, _CPW)], cyw)
    pltpu.sync_copy(nzf.at[pl.ds(cbase, _CPW)], czw)
    boff = b * _N
    lane = lax.iota(jnp.int32, 16)

    def center_body(ci, _):
        ci_splat = jnp.full((16,), ci, jnp.int32)
        cx = plsc.load_gather(cxw, [ci_splat])
        cy = plsc.load_gather(cyw, [ci_splat])
        cz = plsc.load_gather(czw, [ci_splat])

        def cond(st):
            j, cntv = st
            return jnp.logical_and(j < _N // 16, cntv[0] < _NS)

        def chunk(st):
            j, cntv = st
            xv = xw[pl.ds(j * 16, 16)]
            yv = yw[pl.ds(j * 16, 16)]
            zv = zw[pl.ds(j * 16, 16)]
            dx, dy, dz = xv - cx, yv - cy, zv - cz
            d2 = dx * dx + dy * dy + dz * dz
            m = d2 < _R2
            pre = plsc.cumsum(m.astype(jnp.int32))
            slots = cntv + pre - 1
            plsc.store_scatter(buf, [slots], lane + j * 16, mask=m)
            return (j + 1, cntv + plsc.all_reduce_population_count(m))

        _, cntv = lax.while_loop(
            cond, chunk, (jnp.int32(0), jnp.zeros((16,), jnp.int32)))
        vcnt = jnp.minimum(cntv, _NS)
        v0 = buf[pl.ds(0, 16)]
        v1 = buf[pl.ds(16, 16)]
        first = jnp.full((16,), v0[0], jnp.int32)
        rowbuf[pl.ds(0, 16)] = jnp.where(lane < vcnt, v0, first) + boff
        rowbuf[pl.ds(16, 16)] = jnp.where(lane + 16 < vcnt, v1, first) + boff
        pltpu.sync_copy(rowbuf, idx_hbm.at[cbase + ci])
        return 0

    lax.fori_loop(0, _CPW, center_body, 0)


_ROWS_PW = (_B * _NP * _NS) // _NW
_CHUNK = 128


@functools.cache
def _build_gather():
  return pl.kernel(
    _gather_body,
    out_type=jax.ShapeDtypeStruct((_B * _NP * _NS, _DP), jnp.float32),
    mesh=_sc_mesh(),
    compiler_params=pltpu.CompilerParams(needs_layout_passes=False),
    scratch_types=[
        pltpu.VMEM((_CHUNK,), jnp.int32),
        pltpu.VMEM((_CHUNK, _DP), jnp.float32),
        pltpu.SemaphoreType.DMA,
    ],
  )


def _gather_body(t_hbm, idx_hbm, g_hbm, idxv, rows, sem):
    wid = lax.axis_index("s") * _NC + lax.axis_index("c")
    base = wid * _ROWS_PW

    def body(k, _):
        o = base + k * _CHUNK
        pltpu.sync_copy(idx_hbm.at[pl.ds(o, _CHUNK)], idxv)
        pltpu.async_copy(t_hbm.at[idxv], rows, sem).wait()
        pltpu.sync_copy(rows, g_hbm.at[pl.ds(o, _CHUNK)])
        return 0

    lax.fori_loop(0, _ROWS_PW // _CHUNK, body, 0)


def _mm_body(g_ref, w_ref, nx_ref, a_ref, o_ref):
    acc = jnp.dot(g_ref[...], w_ref[...], preferred_element_type=jnp.float32)
    corr = jnp.dot(nx_ref[...], a_ref[...], preferred_element_type=jnp.float32)
    o_ref[...] = acc - corr


def _conv(g2, w2, nxp, a2):
    blk = 256
    grid = (_B * _NP) // blk
    return pl.pallas_call(
        _mm_body,
        grid=(grid,),
        in_specs=[
            pl.BlockSpec((blk, _NS * _DP), lambda i: (i, 0)),
            pl.BlockSpec((_NS * _DP, _COUT), lambda i: (0, 0)),
            pl.BlockSpec((blk, 128), lambda i: (i, 0)),
            pl.BlockSpec((128, _COUT), lambda i: (0, 0)),
        ],
        out_specs=pl.BlockSpec((blk, _COUT), lambda i: (i, 0)),
        out_shape=jax.ShapeDtypeStruct((_B * _NP, _COUT), jnp.float32),
    )(g2, w2, nxp, a2)


def kernel(points, features, W):
    B, N, _ = points.shape
    assert (B, N) == (_B, _N)
    nx, ny, nz = _fps(points)

    pts_t = jnp.transpose(points, (0, 2, 1))
    xf = pts_t[:, 0].reshape(-1)
    yf = pts_t[:, 1].reshape(-1)
    zf = pts_t[:, 2].reshape(-1)
    idx = _build_ballq()(xf, yf, zf,
                         nx.reshape(-1), ny.reshape(-1), nz.reshape(-1))

    table = jnp.concatenate(
        [points, features, jnp.zeros((B, N, _DP - 3 - _CIN), jnp.float32)],
        axis=-1).reshape(B * N, _DP)
    g = _build_gather()(table, idx.reshape(-1))

    wm = W[:, :, 0, :]
    w2 = jnp.pad(jnp.transpose(wm, (2, 1, 0)),
                 ((0, 0), (0, _DP - 3 - _CIN), (0, 0))).reshape(_NS * _DP, _COUT)
    a2 = jnp.pad(jnp.sum(wm[:, :3, :], axis=-1).T, ((0, 125), (0, 0)))
    new_xyz = jnp.stack([nx, ny, nz], axis=-1)
    nxp = jnp.pad(new_xyz.reshape(B * _NP, 3), ((0, 0), (0, 125)))

    conv = _conv(g.reshape(B * _NP, _NS * _DP), w2, nxp, a2)
    return (new_xyz, conv.reshape(B, _NP, _COUT))


# speedup vs baseline: 6.6487x; 1.0040x over previous
import functools

import jax
import jax.numpy as jnp
from jax import lax
from jax.experimental import pallas as pl
from jax.experimental.pallas import tpu as pltpu
from jax.experimental.pallas import tpu_sc as plsc

_R2 = 0.2 * 0.2
_NS = 32
_N = 4096
_NP = 1024
_B = 2
_CIN = 64
_COUT = 64
_DP = 128
_NC, _NSUB = 2, 16
_NW = _NC * _NSUB
_CPW = (_B * _NP) // _NW


def _fps_body(x0_ref, y0_ref, z0_ref, x1_ref, y1_ref, z1_ref,
              nx_ref, ny_ref, nz_ref):
    row = lax.broadcasted_iota(jnp.int32, (32, 128), 0)
    col = lax.broadcasted_iota(jnp.int32, (32, 128), 1)
    gid = row * 128 + col
    col1 = lax.broadcasted_iota(jnp.int32, (1, 128), 1)
    planes = ((x0_ref[...], y0_ref[...], z0_ref[...]),
              (x1_ref[...], y1_ref[...], z1_ref[...]))
    outs = (nx_ref, ny_ref, nz_ref)

    def pick(arr, sel):
        return jnp.sum(jnp.where(sel, arr, 0.0), axis=(0, 1), keepdims=True)

    acc0 = jnp.zeros((1, 128), jnp.float32)

    def outer(r, ocarry):
        d0, d1, cs = ocarry

        def inner(k, carry):
            d0, d1, cs, a = carry
            oh = col1 == k
            accs, newd, newc = list(a), [], []
            for b, d in enumerate((d0, d1)):
                x, y, z = planes[b]
                cx, cy, cz = cs[b * 3], cs[b * 3 + 1], cs[b * 3 + 2]
                for ci, cc in enumerate((cx, cy, cz)):
                    accs[b * 3 + ci] = jnp.where(oh, cc, accs[b * 3 + ci])
                dx, dy, dz = x - cx, y - cy, z - cz
                dd = dx * dx + dy * dy + dz * dz
                d = jnp.minimum(d, dd)
                m = jnp.max(d, axis=(0, 1), keepdims=True)
                cand = jnp.where(d == m, gid, _N)
                l = jnp.min(cand, axis=(0, 1), keepdims=True)
                sel = cand == l
                newd.append(d)
                newc += [pick(x, sel), pick(y, sel), pick(z, sel)]
            return (newd[0], newd[1], tuple(newc), tuple(accs))

        d0, d1, cs, accs = lax.fori_loop(
            0, 128, inner, (d0, d1, cs, (acc0,) * 6))
        for b in range(_B):
            for ci in range(3):
                outs[ci][b, pl.ds(r, 1), :] = accs[b * 3 + ci]
        return (d0, d1, cs)

    sel0 = gid == 0
    cs0 = []
    for b in range(_B):
        x, y, z = planes[b]
        cs0 += [pick(x, sel0), pick(y, sel0), pick(z, sel0)]
    init = (jnp.full((32, 128), 1e10, jnp.float32),
            jnp.full((32, 128), 1e10, jnp.float32), tuple(cs0))
    lax.fori_loop(0, _NP // 128, outer, init)


def _fps(points):
    pts = jnp.transpose(points, (0, 2, 1))
    blocks = [pts[b, d].reshape(32, 128) for b in range(_B) for d in range(3)]
    args = [blocks[0], blocks[1], blocks[2], blocks[3], blocks[4], blocks[5]]
    out = pl.pallas_call(
        _fps_body,
        out_shape=[jax.ShapeDtypeStruct((_B, 8, 128), jnp.float32)] * 3,
        in_specs=[pl.BlockSpec(memory_space=pltpu.VMEM)] * 6,
        out_specs=[pl.BlockSpec(memory_space=pltpu.VMEM)] * 3,
    )(*args)
    return tuple(o.reshape(_B, _NP) for o in out)


@functools.cache
def _sc_mesh():
    return plsc.VectorSubcoreMesh(core_axis_name="c", subcore_axis_name="s",
                                  num_cores=_NC, num_subcores=_NSUB)


@functools.cache
def _build_ballq():
  return pl.kernel(
    _ballq_body,
    out_type=jax.ShapeDtypeStruct((_B * _NP, _NS), jnp.int32),
    mesh=_sc_mesh(),
    compiler_params=pltpu.CompilerParams(needs_layout_passes=False),
    scratch_types=[
        pltpu.VMEM((_N,), jnp.float32),
        pltpu.VMEM((_N,), jnp.float32),
        pltpu.VMEM((_N,), jnp.float32),
        pltpu.VMEM((_CPW,), jnp.float32),
        pltpu.VMEM((_CPW,), jnp.float32),
        pltpu.VMEM((_CPW,), jnp.float32),
        pltpu.VMEM((48,), jnp.int32),
        pltpu.VMEM((_NS,), jnp.int32),
    ],
  )


def _ballq_body(xf, yf, zf, nxf, nyf, nzf, idx_hbm,
                xw, yw, zw, cxw, cyw, czw, buf, rowbuf):
    wid = lax.axis_index("s") * _NC + lax.axis_index("c")
    b = wid // _NSUB
    cbase = wid * _CPW
    pltpu.sync_copy(xf.at[pl.ds(b * _N, _N)], xw)
    pltpu.sync_copy(yf.at[pl.ds(b * _N, _N)], yw)
    pltpu.sync_copy(zf.at[pl.ds(b * _N, _N)], zw)
    pltpu.sync_copy(nxf.at[pl.ds(cbase, _CPW)], cxw)
    pltpu.sync_copy(nyf.at[pl.ds(cbase, _CPW)], cyw)
    pltpu.sync_copy(nzf.at[pl.ds(cbase, _CPW)], czw)
    boff = b * _N
    lane = lax.iota(jnp.int32, 16)

    def center_body(ci, _):
        ci_splat = jnp.full((16,), ci, jnp.int32)
        cx = plsc.load_gather(cxw, [ci_splat])
        cy = plsc.load_gather(cyw, [ci_splat])
        cz = plsc.load_gather(czw, [ci_splat])

        def cond(st):
            j, cntv = st
            return jnp.logical_and(j < _N // 16, cntv[0] < _NS)

        def chunk(st):
            j, cntv = st
            xv = xw[pl.ds(j * 16, 16)]
            yv = yw[pl.ds(j * 16, 16)]
            zv = zw[pl.ds(j * 16, 16)]
            dx, dy, dz = xv - cx, yv - cy, zv - cz
            d2 = dx * dx + dy * dy + dz * dz
            m = d2 < _R2
            pre = plsc.cumsum(m.astype(jnp.int32))
            slots = cntv + pre - 1
            plsc.store_scatter(buf, [slots], lane + j * 16, mask=m)
            return (j + 1, cntv + plsc.all_reduce_population_count(m))

        _, cntv = lax.while_loop(
            cond, chunk, (jnp.int32(0), jnp.zeros((16,), jnp.int32)))
        vcnt = jnp.minimum(cntv, _NS)
        v0 = buf[pl.ds(0, 16)]
        v1 = buf[pl.ds(16, 16)]
        first = jnp.full((16,), v0[0], jnp.int32)
        rowbuf[pl.ds(0, 16)] = jnp.where(lane < vcnt, v0, first) + boff
        rowbuf[pl.ds(16, 16)] = jnp.where(lane + 16 < vcnt, v1, first) + boff
        pltpu.sync_copy(rowbuf, idx_hbm.at[cbase + ci])
        return 0

    lax.fori_loop(0, _CPW, center_body, 0)


_ROWS_PW = (_B * _NP * _NS) // _NW
_CHUNK = 128


@functools.cache
def _build_gather():
  return pl.kernel(
    _gather_body,
    out_type=jax.ShapeDtypeStruct((_B * _NP * _NS, _DP), jnp.float32),
    mesh=_sc_mesh(),
    compiler_params=pltpu.CompilerParams(needs_layout_passes=False),
    scratch_types=[
        pltpu.VMEM((_CHUNK,), jnp.int32),
        pltpu.VMEM((_CHUNK, _DP), jnp.float32),
        pltpu.SemaphoreType.DMA,
    ],
  )


def _gather_body(t_hbm, idx_hbm, g_hbm, idxv, rows, sem):
    wid = lax.axis_index("s") * _NC + lax.axis_index("c")
    base = wid * _ROWS_PW

    def body(k, _):
        o = base + k * _CHUNK
        pltpu.sync_copy(idx_hbm.at[pl.ds(o, _CHUNK)], idxv)
        pltpu.async_copy(t_hbm.at[idxv], rows, sem).wait()
        pltpu.sync_copy(rows, g_hbm.at[pl.ds(o, _CHUNK)])
        return 0

    lax.fori_loop(0, _ROWS_PW // _CHUNK, body, 0)


def _mm_body(g_ref, w_ref, nx_ref, a_ref, o_ref):
    acc = jnp.dot(g_ref[...], w_ref[...], preferred_element_type=jnp.float32)
    corr = jnp.dot(nx_ref[...], a_ref[...], preferred_element_type=jnp.float32)
    o_ref[...] = acc - corr


def _conv(g2, w2, nxp, a2):
    blk = 256
    grid = (_B * _NP) // blk
    return pl.pallas_call(
        _mm_body,
        grid=(grid,),
        in_specs=[
            pl.BlockSpec((blk, _NS * _DP), lambda i: (i, 0)),
            pl.BlockSpec((_NS * _DP, _COUT), lambda i: (0, 0)),
            pl.BlockSpec((blk, 128), lambda i: (i, 0)),
            pl.BlockSpec((128, _COUT), lambda i: (0, 0)),
        ],
        out_specs=pl.BlockSpec((blk, _COUT), lambda i: (i, 0)),
        out_shape=jax.ShapeDtypeStruct((_B * _NP, _COUT), jnp.float32),
    )(g2, w2, nxp, a2)


def kernel(points, features, W):
    B, N, _ = points.shape
    assert (B, N) == (_B, _N)
    nx, ny, nz = _fps(points)

    pts_t = jnp.transpose(points, (0, 2, 1))
    xf = pts_t[:, 0].reshape(-1)
    yf = pts_t[:, 1].reshape(-1)
    zf = pts_t[:, 2].reshape(-1)
    idx = _build_ballq()(xf, yf, zf,
                         nx.reshape(-1), ny.reshape(-1), nz.reshape(-1))

    table = jnp.concatenate(
        [points, features, jnp.zeros((B, N, _DP - 3 - _CIN), jnp.float32)],
        axis=-1).reshape(B * N, _DP)
    g = _build_gather()(table, idx.reshape(-1))

    wm = W[:, :, 0, :]
    w2 = jnp.pad(jnp.transpose(wm, (2, 1, 0)),
                 ((0, 0), (0, _DP - 3 - _CIN), (0, 0))).reshape(_NS * _DP, _COUT)
    a2 = jnp.pad(jnp.sum(wm[:, :3, :], axis=-1).T, ((0, 125), (0, 0)))
    new_xyz = jnp.stack([nx, ny, nz], axis=-1)
    nxp = jnp.pad(new_xyz.reshape(B * _NP, 3), ((0, 0), (0, 125)))

    conv = _conv(g.reshape(B * _NP, _NS * _DP), w2, nxp, a2)
    return (new_xyz, conv.reshape(B, _NP, _COUT))


# speedup vs baseline: 7.7031x; 1.1586x over previous
import functools

import jax
import jax.numpy as jnp
from jax import lax
from jax.experimental import pallas as pl
from jax.experimental.pallas import tpu as pltpu
from jax.experimental.pallas import tpu_sc as plsc

_R2 = 0.2 * 0.2
_NS = 32
_N = 4096
_NP = 1024
_B = 2
_CIN = 64
_COUT = 64
_DP = 128
_NC, _NSUB = 2, 16
_NW = _NC * _NSUB
_CPW = (_B * _NP) // _NW


def _fps_body(p0_ref, p1_ref, out_ref):
    row = lax.broadcasted_iota(jnp.int32, (32, 128), 0)
    col = lax.broadcasted_iota(jnp.int32, (32, 128), 1)
    gid = row * 128 + col
    col1 = lax.broadcasted_iota(jnp.int32, (1, 128), 1)
    refs = (p0_ref, p1_ref)

    def pick(p3, sel):
        return jnp.sum(jnp.where(sel[None], p3, 0.0), axis=(1, 2),
                       keepdims=True)

    acc0 = jnp.zeros((3, 1, 128), jnp.float32)

    def outer(r, ocarry):
        d0, d1, c0, c1 = ocarry

        def inner(k, carry):
            d0, d1, c0, c1, a0, a1 = carry
            oh = (col1 == k)[None]
            new = []
            for d, cc, acc, ref in ((d0, c0, a0, refs[0]),
                                    (d1, c1, a1, refs[1])):
                p3 = ref[...]
                acc = jnp.where(oh, cc, acc)
                dx = p3 - cc
                sq = dx * dx
                dd = (sq[0] + sq[1]) + sq[2]
                d = jnp.minimum(d, dd)
                m = jnp.max(d, axis=(0, 1), keepdims=True)
                cand = jnp.where(d == m, gid, _N)
                l = jnp.min(cand, axis=(0, 1), keepdims=True)
                new.append((d, pick(p3, cand == l), acc))
            return (new[0][0], new[1][0], new[0][1], new[1][1],
                    new[0][2], new[1][2])

        d0, d1, c0, c1, a0, a1 = lax.fori_loop(
            0, 128, inner, (d0, d1, c0, c1, acc0, acc0))
        out_ref[0, :, pl.ds(r, 1), :] = a0
        out_ref[1, :, pl.ds(r, 1), :] = a1
        return (d0, d1, c0, c1)

    sel0 = gid == 0
    dinit = jnp.full((32, 128), 1e10, jnp.float32)
    init = (dinit, dinit, pick(p0_ref[...], sel0), pick(p1_ref[...], sel0))
    lax.fori_loop(0, _NP // 128, outer, init)


def _fps(points):
    pts = jnp.transpose(points, (0, 2, 1))
    p3 = pts.reshape(_B, 3, 32, 128)
    out = pl.pallas_call(
        _fps_body,
        out_shape=jax.ShapeDtypeStruct((_B, 3, 8, 128), jnp.float32),
        in_specs=[pl.BlockSpec(memory_space=pltpu.VMEM)] * 2,
        out_specs=pl.BlockSpec(memory_space=pltpu.VMEM),
    )(p3[0], p3[1])
    coords = out.reshape(_B, 3, _NP)
    return coords[:, 0], coords[:, 1], coords[:, 2]


@functools.cache
def _sc_mesh():
    return plsc.VectorSubcoreMesh(core_axis_name="c", subcore_axis_name="s",
                                  num_cores=_NC, num_subcores=_NSUB)


@functools.cache
def _build_ballq():
  return pl.kernel(
    _ballq_body,
    out_type=jax.ShapeDtypeStruct((_B * _NP, _NS), jnp.int32),
    mesh=_sc_mesh(),
    compiler_params=pltpu.CompilerParams(needs_layout_passes=False),
    scratch_types=[
        pltpu.VMEM((_N,), jnp.float32),
        pltpu.VMEM((_N,), jnp.float32),
        pltpu.VMEM((_N,), jnp.float32),
        pltpu.VMEM((_CPW,), jnp.float32),
        pltpu.VMEM((_CPW,), jnp.float32),
        pltpu.VMEM((_CPW,), jnp.float32),
        pltpu.VMEM((48,), jnp.int32),
        pltpu.VMEM((_NS,), jnp.int32),
    ],
  )


def _ballq_body(xf, yf, zf, nxf, nyf, nzf, idx_hbm,
                xw, yw, zw, cxw, cyw, czw, buf, rowbuf):
    wid = lax.axis_index("s") * _NC + lax.axis_index("c")
    b = wid // _NSUB
    cbase = wid * _CPW
    pltpu.sync_copy(xf.at[pl.ds(b * _N, _N)], xw)
    pltpu.sync_copy(yf.at[pl.ds(b * _N, _N)], yw)
    pltpu.sync_copy(zf.at[pl.ds(b * _N, _N)], zw)
    pltpu.sync_copy(nxf.at[pl.ds(cbase, _CPW)], cxw)
    pltpu.sync_copy(nyf.at[pl.ds(cbase, _CPW)], cyw)
    pltpu.sync_copy(nzf.at[pl.ds(cbase, _CPW)], czw)
    boff = b * _N
    lane = lax.iota(jnp.int32, 16)

    def center_body(ci, _):
        ci_splat = jnp.full((16,), ci, jnp.int32)
        cx = plsc.load_gather(cxw, [ci_splat])
        cy = plsc.load_gather(cyw, [ci_splat])
        cz = plsc.load_gather(czw, [ci_splat])

        def cond(st):
            j, cntv = st
            return jnp.logical_and(j < _N // 16, cntv[0] < _NS)

        def chunk(st):
            j, cntv = st
            xv = xw[pl.ds(j * 16, 16)]
            yv = yw[pl.ds(j * 16, 16)]
            zv = zw[pl.ds(j * 16, 16)]
            dx, dy, dz = xv - cx, yv - cy, zv - cz
            d2 = dx * dx + dy * dy + dz * dz
            m = d2 < _R2
            pre = plsc.cumsum(m.astype(jnp.int32))
            slots = cntv + pre - 1
            plsc.store_scatter(buf, [slots], lane + j * 16, mask=m)
            return (j + 1, cntv + plsc.all_reduce_population_count(m))

        _, cntv = lax.while_loop(
            cond, chunk, (jnp.int32(0), jnp.zeros((16,), jnp.int32)))
        vcnt = jnp.minimum(cntv, _NS)
        v0 = buf[pl.ds(0, 16)]
        v1 = buf[pl.ds(16, 16)]
        first = jnp.full((16,), v0[0], jnp.int32)
        rowbuf[pl.ds(0, 16)] = jnp.where(lane < vcnt, v0, first) + boff
        rowbuf[pl.ds(16, 16)] = jnp.where(lane + 16 < vcnt, v1, first) + boff
        pltpu.sync_copy(rowbuf, idx_hbm.at[cbase + ci])
        return 0

    lax.fori_loop(0, _CPW, center_body, 0)


_ROWS_PW = (_B * _NP * _NS) // _NW
_CHUNK = 128


@functools.cache
def _build_gather():
  return pl.kernel(
    _gather_body,
    out_type=jax.ShapeDtypeStruct((_B * _NP * _NS, _DP), jnp.float32),
    mesh=_sc_mesh(),
    compiler_params=pltpu.CompilerParams(needs_layout_passes=False),
    scratch_types=[
        pltpu.VMEM((_CHUNK,), jnp.int32),
        pltpu.VMEM((_CHUNK, _DP), jnp.float32),
        pltpu.SemaphoreType.DMA,
    ],
  )


def _gather_body(t_hbm, idx_hbm, g_hbm, idxv, rows, sem):
    wid = lax.axis_index("s") * _NC + lax.axis_index("c")
    base = wid * _ROWS_PW

    def body(k, _):
        o = base + k * _CHUNK
        pltpu.sync_copy(idx_hbm.at[pl.ds(o, _CHUNK)], idxv)
        pltpu.async_copy(t_hbm.at[idxv], rows, sem).wait()
        pltpu.sync_copy(rows, g_hbm.at[pl.ds(o, _CHUNK)])
        return 0

    lax.fori_loop(0, _ROWS_PW // _CHUNK, body, 0)


def _mm_body(g_ref, w_ref, nx_ref, a_ref, o_ref):
    acc = jnp.dot(g_ref[...], w_ref[...], preferred_element_type=jnp.float32)
    corr = jnp.dot(nx_ref[...], a_ref[...], preferred_element_type=jnp.float32)
    o_ref[...] = acc - corr


def _conv(g2, w2, nxp, a2):
    blk = 256
    grid = (_B * _NP) // blk
    return pl.pallas_call(
        _mm_body,
        grid=(grid,),
        in_specs=[
            pl.BlockSpec((blk, _NS * _DP), lambda i: (i, 0)),
            pl.BlockSpec((_NS * _DP, _COUT), lambda i: (0, 0)),
            pl.BlockSpec((blk, 128), lambda i: (i, 0)),
            pl.BlockSpec((128, _COUT), lambda i: (0, 0)),
        ],
        out_specs=pl.BlockSpec((blk, _COUT), lambda i: (i, 0)),
        out_shape=jax.ShapeDtypeStruct((_B * _NP, _COUT), jnp.float32),
    )(g2, w2, nxp, a2)


def kernel(points, features, W):
    B, N, _ = points.shape
    assert (B, N) == (_B, _N)
    nx, ny, nz = _fps(points)

    pts_t = jnp.transpose(points, (0, 2, 1))
    xf = pts_t[:, 0].reshape(-1)
    yf = pts_t[:, 1].reshape(-1)
    zf = pts_t[:, 2].reshape(-1)
    idx = _build_ballq()(xf, yf, zf,
                         nx.reshape(-1), ny.reshape(-1), nz.reshape(-1))

    table = jnp.concatenate(
        [points, features, jnp.zeros((B, N, _DP - 3 - _CIN), jnp.float32)],
        axis=-1).reshape(B * N, _DP)
    g = _build_gather()(table, idx.reshape(-1))

    wm = W[:, :, 0, :]
    w2 = jnp.pad(jnp.transpose(wm, (2, 1, 0)),
                 ((0, 0), (0, _DP - 3 - _CIN), (0, 0))).reshape(_NS * _DP, _COUT)
    a2 = jnp.pad(jnp.sum(wm[:, :3, :], axis=-1).T, ((0, 125), (0, 0)))
    new_xyz = jnp.stack([nx, ny, nz], axis=-1)
    nxp = jnp.pad(new_xyz.reshape(B * _NP, 3), ((0, 0), (0, 125)))

    conv = _conv(g.reshape(B * _NP, _NS * _DP), w2, nxp, a2)
    return (new_xyz, conv.reshape(B, _NP, _COUT))
